# Initial kernel scaffold; baseline (speedup 1.0000x reference)
import jax
import jax.numpy as jnp
from jax.experimental import pallas as pl


def kernel(x, edge_index, W1, b1, W2, b2):
    raise NotImplementedError("write your pallas kernel here")


# speedup vs baseline: 21.8919x; 21.8919x over previous
import functools

import jax
import jax.numpy as jnp
from jax import lax
from jax.experimental import pallas as pl
from jax.experimental.pallas import tpu as pltpu
from jax.experimental.pallas import tpu_sc as plsc

N = 10000
D = 128
NPAD = 10240
NW = 32
CH = 128
CHUNKS = 80
EPAD = NW * CHUNKS * CH
RPT = NPAD // 16

_MESH = plsc.VectorSubcoreMesh(core_axis_name="c", subcore_axis_name="s")


def _deg_body(dst_hbm, degp_hbm, idx_d, buf, acc):
    cid = lax.axis_index("c")
    sid = lax.axis_index("s")
    wid = cid * 16 + sid
    zeros = jnp.zeros((16,), jnp.float32)
    ones = jnp.full((16,), 1.0, jnp.float32)
    for j in range(8):
        buf[0, pl.ds(j * 16, 16)] = zeros
        buf[1, pl.ds(j * 16, 16)] = ones
    for k in range(RPT // CH):
        pltpu.sync_copy(buf.at[0], acc.at[pl.ds(sid * RPT + k * CH, CH)])
    plsc.subcore_barrier()
    pltpu.sync_copy(dst_hbm.at[wid], idx_d)

    def body(c, carry):
        pltpu.sync_copy(buf.at[1], acc.at[idx_d.at[c]], add=True)
        return carry

    lax.fori_loop(0, CHUNKS, body, 0)
    plsc.subcore_barrier()
    pltpu.sync_copy(acc.at[pl.ds(sid * RPT, RPT)],
                    degp_hbm.at[cid, pl.ds(sid * RPT, RPT)])


_deg_call = pl.kernel(
    _deg_body,
    out_type=jax.ShapeDtypeStruct((2, NPAD), jnp.float32),
    mesh=_MESH,
    scratch_types=[
        pltpu.VMEM((CHUNKS, CH), jnp.int32),
        pltpu.VMEM((2, CH), jnp.float32),
        pltpu.VMEM_SHARED((NPAD,), jnp.float32),
    ],
)


def _agg_body(y_hbm, src_hbm, dst_hbm, ap_hbm, idx_s, idx_d, rows, acc, gsem):
    cid = lax.axis_index("c")
    sid = lax.axis_index("s")
    wid = cid * 16 + sid
    zeros = jnp.zeros((16,), jnp.float32)

    def zbody(i, carry):
        for j in range(8):
            rows[i, pl.ds(j * 16, 16)] = zeros
        return carry

    lax.fori_loop(0, CH, zbody, 0)
    for k in range(RPT // CH):
        pltpu.sync_copy(rows, acc.at[pl.ds(sid * RPT + k * CH, CH)])
    plsc.subcore_barrier()
    pltpu.sync_copy(src_hbm.at[wid], idx_s)
    pltpu.sync_copy(dst_hbm.at[wid], idx_d)

    def body(c, carry):
        pltpu.async_copy(y_hbm.at[idx_s.at[c]], rows, gsem).wait()
        pltpu.sync_copy(rows, acc.at[idx_d.at[c]], add=True)
        return carry

    lax.fori_loop(0, CHUNKS, body, 0)
    plsc.subcore_barrier()
    pltpu.sync_copy(acc.at[pl.ds(sid * RPT, RPT)],
                    ap_hbm.at[cid, pl.ds(sid * RPT, RPT)])


_agg_call = pl.kernel(
    _agg_body,
    out_type=jax.ShapeDtypeStruct((2, NPAD, D), jnp.float32),
    mesh=_MESH,
    scratch_types=[
        pltpu.VMEM((CHUNKS, CH), jnp.int32),
        pltpu.VMEM((CHUNKS, CH), jnp.int32),
        pltpu.VMEM((CH, D), jnp.float32),
        pltpu.VMEM_SHARED((NPAD, D), jnp.float32),
        pltpu.SemaphoreType.DMA,
    ],
)


def _dinv_of(degt_ref):
    deg = degt_ref[:, 0:1] + degt_ref[:, 1:2] + 1.0
    return lax.rsqrt(deg)


def _tc1_body(x_ref, w_ref, degt_ref, y_ref, xw_ref):
    xw = jnp.dot(x_ref[...], w_ref[...], preferred_element_type=jnp.float32)
    dinv = _dinv_of(degt_ref)
    xw_ref[...] = xw
    y_ref[...] = xw * dinv


def _tc2_body(ap_ref, degt_ref, xw1_ref, w2_ref, b1_ref, y2_ref, xw2_ref):
    dinv = _dinv_of(degt_ref)
    agg = ap_ref[0] + ap_ref[1]
    h = jnp.maximum(dinv * agg + (dinv * dinv) * xw1_ref[...] + b1_ref[...], 0.0)
    xw2 = jnp.dot(h, w2_ref[...], preferred_element_type=jnp.float32)
    xw2_ref[...] = xw2
    y2_ref[...] = xw2 * dinv


def _tc3_body(ap_ref, degt_ref, xw2_ref, b2_ref, out_ref):
    dinv = _dinv_of(degt_ref)
    agg = ap_ref[0] + ap_ref[1]
    out_ref[...] = dinv * agg + (dinv * dinv) * xw2_ref[...] + b2_ref[...]


_BLK = 640
_f32 = jnp.float32


def _tc1(x_p, W1, degt):
    return pl.pallas_call(
        _tc1_body,
        grid=(NPAD // _BLK,),
        in_specs=[
            pl.BlockSpec((_BLK, D), lambda i: (i, 0)),
            pl.BlockSpec((D, D), lambda i: (0, 0)),
            pl.BlockSpec((_BLK, 2), lambda i: (i, 0)),
        ],
        out_specs=[
            pl.BlockSpec((_BLK, D), lambda i: (i, 0)),
            pl.BlockSpec((_BLK, D), lambda i: (i, 0)),
        ],
        out_shape=[
            jax.ShapeDtypeStruct((NPAD, D), _f32),
            jax.ShapeDtypeStruct((NPAD, D), _f32),
        ],
    )(x_p, W1, degt)


def _tc2(ap1, degt, xw1, W2, b1):
    return pl.pallas_call(
        _tc2_body,
        grid=(NPAD // _BLK,),
        in_specs=[
            pl.BlockSpec((2, _BLK, D), lambda i: (0, i, 0)),
            pl.BlockSpec((_BLK, 2), lambda i: (i, 0)),
            pl.BlockSpec((_BLK, D), lambda i: (i, 0)),
            pl.BlockSpec((D, D), lambda i: (0, 0)),
            pl.BlockSpec((1, D), lambda i: (0, 0)),
        ],
        out_specs=[
            pl.BlockSpec((_BLK, D), lambda i: (i, 0)),
            pl.BlockSpec((_BLK, D), lambda i: (i, 0)),
        ],
        out_shape=[
            jax.ShapeDtypeStruct((NPAD, D), _f32),
            jax.ShapeDtypeStruct((NPAD, D), _f32),
        ],
    )(ap1, degt, xw1, W2, b1)


def _tc3(ap2, degt, xw2, b2):
    return pl.pallas_call(
        _tc3_body,
        grid=(NPAD // _BLK,),
        in_specs=[
            pl.BlockSpec((2, _BLK, D), lambda i: (0, i, 0)),
            pl.BlockSpec((_BLK, 2), lambda i: (i, 0)),
            pl.BlockSpec((_BLK, D), lambda i: (i, 0)),
            pl.BlockSpec((1, D), lambda i: (0, 0)),
        ],
        out_specs=pl.BlockSpec((_BLK, D), lambda i: (i, 0)),
        out_shape=jax.ShapeDtypeStruct((NPAD, D), _f32),
    )(ap2, degt, xw2, b2)


def kernel(x, edge_index, W1, b1, W2, b2):
    src = edge_index[0]
    dst = edge_index[1]
    e = src.shape[0]
    padn = EPAD - e
    padidx = (jnp.arange(padn, dtype=jnp.int32) % (NPAD - N)) + N
    src_p = jnp.concatenate([src.astype(jnp.int32), padidx]).reshape(NW, CHUNKS, CH)
    dst_p = jnp.concatenate([dst.astype(jnp.int32), padidx]).reshape(NW, CHUNKS, CH)
    x_p = jnp.concatenate([x, jnp.zeros((NPAD - N, D), x.dtype)])

    degp = _deg_call(dst_p)
    degt = degp.T
    y1, xw1 = _tc1(x_p, W1, degt)
    ap1 = _agg_call(y1, src_p, dst_p)
    y2, xw2 = _tc2(ap1, degt, xw1, W2, b1.reshape(1, D))
    ap2 = _agg_call(y2, src_p, dst_p)
    out = _tc3(ap2, degt, xw2, b2.reshape(1, D))
    return out[:N]


# speedup vs baseline: 31.1887x; 1.4247x over previous
import functools

import jax
import jax.numpy as jnp
from jax import lax
from jax.experimental import pallas as pl
from jax.experimental.pallas import tpu as pltpu
from jax.experimental.pallas import tpu_sc as plsc

N = 10000
D = 128
NPAD = 10240
NW = 32
CH = 128
CHUNKS = 80
EPAD = NW * CHUNKS * CH
RPT = NPAD // 16
IH = 2
HC = CHUNKS // IH

_MESH = plsc.VectorSubcoreMesh(core_axis_name="c", subcore_axis_name="s")


def _deg_body(dst_hbm, degp_hbm, idx_d, buf, acc):
    cid = lax.axis_index("c")
    sid = lax.axis_index("s")
    wid = cid * 16 + sid
    zeros = jnp.zeros((16,), jnp.float32)
    ones = jnp.full((16,), 1.0, jnp.float32)
    for j in range(8):
        buf[0, pl.ds(j * 16, 16)] = zeros
        buf[1, pl.ds(j * 16, 16)] = ones
    for k in range(RPT // CH):
        pltpu.sync_copy(buf.at[0], acc.at[pl.ds(sid * RPT + k * CH, CH)])
    plsc.subcore_barrier()
    pltpu.sync_copy(dst_hbm.at[wid], idx_d)

    def body(c, carry):
        pltpu.sync_copy(buf.at[1], acc.at[idx_d.at[c]], add=True)
        return carry

    lax.fori_loop(0, CHUNKS, body, 0)
    plsc.subcore_barrier()
    pltpu.sync_copy(acc.at[pl.ds(sid * RPT, RPT)],
                    degp_hbm.at[cid, pl.ds(sid * RPT, RPT)])


_deg_call = pl.kernel(
    _deg_body,
    out_type=jax.ShapeDtypeStruct((2, NPAD), jnp.float32),
    mesh=_MESH,
    scratch_types=[
        pltpu.VMEM((CHUNKS, CH), jnp.int32),
        pltpu.VMEM((2, CH), jnp.float32),
        pltpu.VMEM_SHARED((NPAD,), jnp.float32),
    ],
)


def _agg_body(y_hbm, src_hbm, dst_hbm, ap_hbm, idx_s, idx_d, rows_a, rows_b,
              acc, sem_a, sem_b):
    cid = lax.axis_index("c")
    sid = lax.axis_index("s")
    wid = cid * 16 + sid
    zeros = jnp.zeros((16,), jnp.float32)

    def zbody(i, carry):
        for j in range(8):
            rows_a[i, pl.ds(j * 16, 16)] = zeros
        return carry

    lax.fori_loop(0, CH, zbody, 0)
    for k in range(RPT // CH):
        pltpu.sync_copy(rows_a, acc.at[pl.ds(sid * RPT + k * CH, CH)])
    plsc.subcore_barrier()

    for h in range(IH):
        pltpu.sync_copy(src_hbm.at[wid, pl.ds(h * HC, HC)], idx_s)
        pltpu.sync_copy(dst_hbm.at[wid, pl.ds(h * HC, HC)], idx_d)

        pltpu.async_copy(y_hbm.at[idx_s.at[0]], rows_a, sem_a)
        pltpu.async_copy(y_hbm.at[idx_s.at[1]], rows_b, sem_b)

        def body(i, carry):
            c = i * 2
            pltpu.make_async_copy(y_hbm.at[idx_s.at[c]], rows_a, sem_a).wait()
            pltpu.sync_copy(rows_a, acc.at[idx_d.at[c]], add=True)

            @pl.when(c + 2 < HC)
            def _():
                pltpu.async_copy(y_hbm.at[idx_s.at[c + 2]], rows_a, sem_a)

            pltpu.make_async_copy(y_hbm.at[idx_s.at[c + 1]], rows_b, sem_b).wait()
            pltpu.sync_copy(rows_b, acc.at[idx_d.at[c + 1]], add=True)

            @pl.when(c + 3 < HC)
            def _():
                pltpu.async_copy(y_hbm.at[idx_s.at[c + 3]], rows_b, sem_b)

            return carry

        lax.fori_loop(0, HC // 2, body, 0)
    plsc.subcore_barrier()
    pltpu.sync_copy(acc.at[pl.ds(sid * RPT, RPT)],
                    ap_hbm.at[cid, pl.ds(sid * RPT, RPT)])


_agg_call = pl.kernel(
    _agg_body,
    out_type=jax.ShapeDtypeStruct((2, NPAD, D), jnp.float32),
    mesh=_MESH,
    scratch_types=[
        pltpu.VMEM((HC, CH), jnp.int32),
        pltpu.VMEM((HC, CH), jnp.int32),
        pltpu.VMEM((CH, D), jnp.float32),
        pltpu.VMEM((CH, D), jnp.float32),
        pltpu.VMEM_SHARED((NPAD, D), jnp.float32),
        pltpu.SemaphoreType.DMA,
        pltpu.SemaphoreType.DMA,
    ],
)


def _dinv_of(degt_ref):
    deg = degt_ref[:, 0:1] + degt_ref[:, 1:2] + 1.0
    return lax.rsqrt(deg)


def _tc1_body(x_ref, w_ref, degt_ref, y_ref, xw_ref):
    xw = jnp.dot(x_ref[...], w_ref[...], preferred_element_type=jnp.float32)
    dinv = _dinv_of(degt_ref)
    xw_ref[...] = xw
    y_ref[...] = xw * dinv


def _tc2_body(ap_ref, degt_ref, xw1_ref, w2_ref, b1_ref, y2_ref, xw2_ref):
    dinv = _dinv_of(degt_ref)
    agg = ap_ref[0] + ap_ref[1]
    h = jnp.maximum(dinv * agg + (dinv * dinv) * xw1_ref[...] + b1_ref[...], 0.0)
    xw2 = jnp.dot(h, w2_ref[...], preferred_element_type=jnp.float32)
    xw2_ref[...] = xw2
    y2_ref[...] = xw2 * dinv


def _tc3_body(ap_ref, degt_ref, xw2_ref, b2_ref, out_ref):
    dinv = _dinv_of(degt_ref)
    agg = ap_ref[0] + ap_ref[1]
    out_ref[...] = dinv * agg + (dinv * dinv) * xw2_ref[...] + b2_ref[...]


_BLK = 640
_f32 = jnp.float32


def _tc1(x_p, W1, degt):
    return pl.pallas_call(
        _tc1_body,
        grid=(NPAD // _BLK,),
        in_specs=[
            pl.BlockSpec((_BLK, D), lambda i: (i, 0)),
            pl.BlockSpec((D, D), lambda i: (0, 0)),
            pl.BlockSpec((_BLK, 2), lambda i: (i, 0)),
        ],
        out_specs=[
            pl.BlockSpec((_BLK, D), lambda i: (i, 0)),
            pl.BlockSpec((_BLK, D), lambda i: (i, 0)),
        ],
        out_shape=[
            jax.ShapeDtypeStruct((NPAD, D), _f32),
            jax.ShapeDtypeStruct((NPAD, D), _f32),
        ],
    )(x_p, W1, degt)


def _tc2(ap1, degt, xw1, W2, b1):
    return pl.pallas_call(
        _tc2_body,
        grid=(NPAD // _BLK,),
        in_specs=[
            pl.BlockSpec((2, _BLK, D), lambda i: (0, i, 0)),
            pl.BlockSpec((_BLK, 2), lambda i: (i, 0)),
            pl.BlockSpec((_BLK, D), lambda i: (i, 0)),
            pl.BlockSpec((D, D), lambda i: (0, 0)),
            pl.BlockSpec((1, D), lambda i: (0, 0)),
        ],
        out_specs=[
            pl.BlockSpec((_BLK, D), lambda i: (i, 0)),
            pl.BlockSpec((_BLK, D), lambda i: (i, 0)),
        ],
        out_shape=[
            jax.ShapeDtypeStruct((NPAD, D), _f32),
            jax.ShapeDtypeStruct((NPAD, D), _f32),
        ],
    )(ap1, degt, xw1, W2, b1)


def _tc3(ap2, degt, xw2, b2):
    return pl.pallas_call(
        _tc3_body,
        grid=(NPAD // _BLK,),
        in_specs=[
            pl.BlockSpec((2, _BLK, D), lambda i: (0, i, 0)),
            pl.BlockSpec((_BLK, 2), lambda i: (i, 0)),
            pl.BlockSpec((_BLK, D), lambda i: (i, 0)),
            pl.BlockSpec((1, D), lambda i: (0, 0)),
        ],
        out_specs=pl.BlockSpec((_BLK, D), lambda i: (i, 0)),
        out_shape=jax.ShapeDtypeStruct((NPAD, D), _f32),
    )(ap2, degt, xw2, b2)


def kernel(x, edge_index, W1, b1, W2, b2):
    src = edge_index[0]
    dst = edge_index[1]
    e = src.shape[0]
    padn = EPAD - e
    padidx = (jnp.arange(padn, dtype=jnp.int32) % (NPAD - N)) + N
    src_p = jnp.concatenate([src.astype(jnp.int32), padidx]).reshape(NW, CHUNKS, CH)
    dst_p = jnp.concatenate([dst.astype(jnp.int32), padidx]).reshape(NW, CHUNKS, CH)
    x_p = jnp.concatenate([x, jnp.zeros((NPAD - N, D), x.dtype)])

    degp = _deg_call(dst_p)
    degt = degp.T
    y1, xw1 = _tc1(x_p, W1, degt)
    ap1 = _agg_call(y1, src_p, dst_p)
    y2, xw2 = _tc2(ap1, degt, xw1, W2, b1.reshape(1, D))
    ap2 = _agg_call(y2, src_p, dst_p)
    out = _tc3(ap2, degt, xw2, b2.reshape(1, D))
    return out[:N]
